# baseline (speedup 1.0000x reference)
import jax
import jax.numpy as jnp
from jax import lax
from jax.experimental import pallas as pl
from jax.experimental.pallas import tpu as pltpu
from jax.experimental.pallas import tpu_sc as plsc

NUM_EMBEDDINGS = 1000000
D = 32
PADDING_IDX = 0
B = 16384
HIST = 50

L = 16
NC, NS = 2, 16
NW = NC * NS
RW = B // NW
CB = 32
GC = CB * HIST
NCHUNK = RW // CB

SB = 512
NSB = NUM_EMBEDDINGS // SB
TAIL_H = NUM_EMBEDDINGS - NSB * SB
SB_PER_W = 62


def _tbody(tt_hbm, out_hbm, b0, b1, o0, o1, si0, si1, so0, so1, tb, tob):
    wid = lax.axis_index("s") * NC + lax.axis_index("c")
    start = 61 * wid + jnp.minimum(wid, 1)
    cnt = jnp.where(wid < 1, 62, 61)
    bufs = (b0, b1)
    obufs = (o0, o1)
    semi = (si0, si1)
    semo = (so0, so1)
    iota = lax.iota(jnp.int32, L)

    def start_in(i, p):
        H0 = (start + i) * SB
        for t in range(D // 8):
            for j in range(SB // 128):
                pltpu.async_copy(
                    tt_hbm.at[pl.ds(8 * t, 8), pl.ds(H0 + 128 * j, 128)],
                    bufs[p].at[t * (SB // 128) + j], semi[p])

    def wait_in(i, p):
        H0 = (start + i) * SB
        for t in range(D // 8):
            for j in range(SB // 128):
                pltpu.make_async_copy(
                    tt_hbm.at[pl.ds(8 * t, 8), pl.ds(H0 + 128 * j, 128)],
                    bufs[p].at[t * (SB // 128) + j], semi[p]).wait()

    tA = (iota // 8) * (SB // 128)
    tB = tA + 2 * (SB // 128)
    ddA = jnp.bitwise_and(iota, 7)

    def transpose_tiles(buf, obuf):
        def hh_body(hg, c):
            h0 = hg * L
            h0b = h0 * D
            for r in range(L):
                t = jnp.bitwise_and(iota + r, L - 1)
                hh = h0 + t
                j = hh // 128
                h128 = jnp.bitwise_and(hh, 127)
                vA = plsc.load_gather(buf, [tA + j, ddA, h128])
                vB = plsc.load_gather(buf, [tB + j, ddA, h128])
                oA = t * D + iota + h0b
                plsc.store_scatter(obuf, [oA], vA)
                plsc.store_scatter(obuf, [oA + L], vB)
            return c

        lax.fori_loop(0, SB // L, hh_body, 0)

    def transpose_buf(buf, obuf, nh):
        def hh_body(hg, c):
            h0 = hg * L
            h0b = h0 * D
            for r in range(L):
                t = jnp.bitwise_and(iota + r, L - 1)
                hh = h0 + t
                vA = plsc.load_gather(buf, [iota, hh])
                vB = plsc.load_gather(buf, [iota + L, hh])
                oA = t * D + iota + h0b
                plsc.store_scatter(obuf, [oA], vA)
                plsc.store_scatter(obuf, [oA + L], vB)
            return c

        lax.fori_loop(0, nh // L, hh_body, 0)

    start_in(0, 0)

    def outer(ib, c):
        for p in range(2):
            i = 2 * ib + p

            @pl.when(i < cnt)
            def _():
                @pl.when(i + 1 < cnt)
                def _():
                    start_in(i + 1, 1 - p)

                H0 = (start + i) * SB
                wait_in(i, p)

                @pl.when(i >= 2)
                def _():
                    pltpu.make_async_copy(
                        obufs[p], out_hbm.at[pl.ds(0, SB * D)],
                        semo[p]).wait()

                transpose_tiles(bufs[p], obufs[p])
                pltpu.async_copy(obufs[p],
                                 out_hbm.at[pl.ds(H0 * D, SB * D)], semo[p])
        return c

    lax.fori_loop(0, SB_PER_W // 2, outer, 0)
    for p in range(2):
        pltpu.make_async_copy(obufs[p], out_hbm.at[pl.ds(0, SB * D)],
                              semo[p]).wait()

    @pl.when(wid == NW - 1)
    def _():
        pltpu.sync_copy(tt_hbm.at[pl.ds(0, D), pl.ds(NSB * SB, TAIL_H)], tb)
        transpose_buf(tb, tob, TAIL_H)
        pltpu.sync_copy(tob, out_hbm.at[pl.ds(NSB * SB * D, TAIL_H * D)])


def _body(hashes_hbm, wts_hbm, table_hbm, out_hbm,
          idx0, wts0, idx1, wts1, idx2, wts2, rows0, rows1, outb_v,
          semi0, semi1, semi2, sem0, sem1):
    wid = lax.axis_index("s") * NC + lax.axis_index("c")
    idx = (idx0, idx1, idx2)
    wts = (wts0, wts1, wts2)
    semi = (semi0, semi1, semi2)
    rows = (rows0, rows1)
    sem = (sem0, sem1)

    def load_start(g, q):
        base_g = (wid * RW + g * CB) * HIST
        pltpu.async_copy(hashes_hbm.at[pl.ds(base_g, GC)], idx[q], semi[q])
        pltpu.async_copy(wts_hbm.at[pl.ds(base_g, GC)],
                         wts[q].at[pl.ds(0, GC)], semi[q])

    def gather_start(g, q, p):
        base_g = (wid * RW + g * CB) * HIST
        pltpu.make_async_copy(hashes_hbm.at[pl.ds(base_g, GC)], idx[q],
                              semi[q]).wait()
        pltpu.make_async_copy(wts_hbm.at[pl.ds(base_g, GC)],
                              wts[q].at[pl.ds(0, GC)], semi[q]).wait()

        def wm_body(j, c):
            iv = idx[q][pl.ds(j * L, L)]
            wv = wts[q][pl.ds(j * L, L)]
            wts[q][pl.ds(j * L, L)] = jnp.where(iv == PADDING_IDX, 0.0, wv)
            return c

        lax.fori_loop(0, GC // L, wm_body, 0)
        pltpu.async_copy(table_hbm.at[idx[q]], rows[p], sem[p])

    def consume(g, q, p):
        pltpu.make_async_copy(table_hbm.at[idx[q]], rows[p], sem[p]).wait()
        rv, wv = rows[p], wts[q]

        def row_body(b, c):
            r0 = b * HIST

            def k_body(k, acc):
                a0, a1, b0, b1 = acc
                rk = r0 + k * L
                w16 = wv[pl.ds(rk, L)]
                for j in range(0, L, 2):
                    w = w16[j]
                    a0 = a0 + w * rv[rk + j, pl.ds(0, L)]
                    a1 = a1 + w * rv[rk + j, pl.ds(L, L)]
                    w2 = w16[j + 1]
                    b0 = b0 + w2 * rv[rk + j + 1, pl.ds(0, L)]
                    b1 = b1 + w2 * rv[rk + j + 1, pl.ds(L, L)]
                return (a0, a1, b0, b1)

            z = jnp.zeros((L,), jnp.float32)
            a0, a1, b0, b1 = lax.fori_loop(0, HIST // L, k_body, (z, z, z, z))
            rt = r0 + (HIST // L) * L
            wt16 = wv[pl.ds(rt, L)]
            a0 = a0 + wt16[0] * rv[rt, pl.ds(0, L)]
            a1 = a1 + wt16[0] * rv[rt, pl.ds(L, L)]
            b0 = b0 + wt16[1] * rv[rt + 1, pl.ds(0, L)]
            b1 = b1 + wt16[1] * rv[rt + 1, pl.ds(L, L)]
            outb_v[b, pl.ds(0, L)] = a0 + b0
            outb_v[b, pl.ds(L, L)] = a1 + b1
            return c

        lax.fori_loop(0, CB, row_body, 0)
        base_b = wid * RW + g * CB
        pltpu.sync_copy(outb_v, out_hbm.at[pl.ds(base_b, CB)])

    load_start(0, 0)
    load_start(1, 1)
    gather_start(0, 0, 0)

    def outer(gb, c):
        for u in range(6):
            g = 6 * gb + u

            @pl.when(g + 2 < NCHUNK)
            def _():
                load_start(g + 2, (u + 2) % 3)

            @pl.when(g + 1 < NCHUNK)
            def _():
                gather_start(g + 1, (u + 1) % 3, (u + 1) % 2)

            @pl.when(g < NCHUNK)
            def _():
                consume(g, u % 3, u % 2)
        return c

    lax.fori_loop(0, (NCHUNK + 5) // 6, outer, 0)


@jax.jit
def kernel(hashes, weights, table):
    hashes_flat = hashes.astype(jnp.int32).reshape(B * HIST)
    weights_flat = weights.reshape(B * HIST)
    mesh = plsc.VectorSubcoreMesh(core_axis_name="c", subcore_axis_name="s")

    t_flat = pl.kernel(
        _tbody,
        out_type=jax.ShapeDtypeStruct((NUM_EMBEDDINGS * D,), jnp.float32),
        mesh=mesh,
        scratch_types=[
            pltpu.VMEM((D // 8 * (SB // 128), 8, 128), jnp.float32),
            pltpu.VMEM((D // 8 * (SB // 128), 8, 128), jnp.float32),
            pltpu.VMEM((SB * D,), jnp.float32),
            pltpu.VMEM((SB * D,), jnp.float32),
            pltpu.SemaphoreType.DMA,
            pltpu.SemaphoreType.DMA,
            pltpu.SemaphoreType.DMA,
            pltpu.SemaphoreType.DMA,
            pltpu.VMEM((D, TAIL_H), jnp.float32),
            pltpu.VMEM((TAIL_H * D,), jnp.float32),
        ],
        compiler_params=pltpu.CompilerParams(use_tc_tiling_on_sc=True,
                                             needs_layout_passes=False),
    )(table.T)
    t_rm = t_flat.reshape(NUM_EMBEDDINGS, D)

    run = pl.kernel(
        _body,
        out_type=jax.ShapeDtypeStruct((B, D), jnp.float32),
        mesh=mesh,
        scratch_types=[
            pltpu.VMEM((GC,), jnp.int32),
            pltpu.VMEM((GC + L,), jnp.float32),
            pltpu.VMEM((GC,), jnp.int32),
            pltpu.VMEM((GC + L,), jnp.float32),
            pltpu.VMEM((GC,), jnp.int32),
            pltpu.VMEM((GC + L,), jnp.float32),
            pltpu.VMEM((GC, D), jnp.float32),
            pltpu.VMEM((GC, D), jnp.float32),
            pltpu.VMEM((CB, D), jnp.float32),
            pltpu.SemaphoreType.DMA,
            pltpu.SemaphoreType.DMA,
            pltpu.SemaphoreType.DMA,
            pltpu.SemaphoreType.DMA,
            pltpu.SemaphoreType.DMA,
        ],
        compiler_params=pltpu.CompilerParams(use_tc_tiling_on_sc=False),
    )
    return run(hashes_flat, weights_flat, t_rm)


# speedup vs baseline: 1.1081x; 1.1081x over previous
import jax
import jax.numpy as jnp
from jax import lax
from jax.experimental import pallas as pl
from jax.experimental.pallas import tpu as pltpu
from jax.experimental.pallas import tpu_sc as plsc

NUM_EMBEDDINGS = 1000000
D = 32
PADDING_IDX = 0
B = 16384
HIST = 50

L = 16
NC, NS = 2, 16
NW = NC * NS
RW = B // NW
CB = 32
GC = CB * HIST
NCHUNK = RW // CB

SB = 512
NSB = NUM_EMBEDDINGS // SB
TAIL_H = NUM_EMBEDDINGS - NSB * SB
SB_PER_W = 62


def _tbody(tt_hbm, out_hbm, b0, b1, o0, o1, si0, si1, so0, so1, tb, tob):
    wid = lax.axis_index("s") * NC + lax.axis_index("c")
    start = 61 * wid + jnp.minimum(wid, 1)
    cnt = jnp.where(wid < 1, 62, 61)
    bufs = (b0, b1)
    obufs = (o0, o1)
    semi = (si0, si1)
    semo = (so0, so1)
    iota = lax.iota(jnp.int32, L)

    def start_in(i, p):
        H0 = (start + i) * SB
        pltpu.async_copy(tt_hbm.at[pl.ds(0, D), pl.ds(H0, SB)],
                         bufs[p], semi[p])

    def wait_in(i, p):
        H0 = (start + i) * SB
        pltpu.make_async_copy(tt_hbm.at[pl.ds(0, D), pl.ds(H0, SB)],
                              bufs[p], semi[p]).wait()

    def transpose_buf(buf, obuf, nh):
        def hh_body(hg, c):
            h0 = hg * L
            h0b = h0 * D
            for r in range(L):
                t = jnp.bitwise_and(iota + r, L - 1)
                hh = h0 + t
                vA = plsc.load_gather(buf, [iota, hh])
                vB = plsc.load_gather(buf, [iota + L, hh])
                oA = t * D + iota + h0b
                plsc.store_scatter(obuf, [oA], vA)
                plsc.store_scatter(obuf, [oA + L], vB)
            return c

        lax.fori_loop(0, nh // L, hh_body, 0)

    start_in(0, 0)

    def outer(ib, c):
        for p in range(2):
            i = 2 * ib + p

            @pl.when(i < cnt)
            def _():
                @pl.when(i + 1 < cnt)
                def _():
                    start_in(i + 1, 1 - p)

                H0 = (start + i) * SB
                wait_in(i, p)

                @pl.when(i >= 2)
                def _():
                    pltpu.make_async_copy(
                        obufs[p], out_hbm.at[pl.ds(0, SB * D)],
                        semo[p]).wait()

                transpose_buf(bufs[p], obufs[p], SB)
                pltpu.async_copy(obufs[p],
                                 out_hbm.at[pl.ds(H0 * D, SB * D)], semo[p])
        return c

    lax.fori_loop(0, SB_PER_W // 2, outer, 0)
    for p in range(2):
        pltpu.make_async_copy(obufs[p], out_hbm.at[pl.ds(0, SB * D)],
                              semo[p]).wait()

    @pl.when(wid == NW - 1)
    def _():
        pltpu.sync_copy(tt_hbm.at[pl.ds(0, D), pl.ds(NSB * SB, TAIL_H)], tb)
        transpose_buf(tb, tob, TAIL_H)
        pltpu.sync_copy(tob, out_hbm.at[pl.ds(NSB * SB * D, TAIL_H * D)])


def _body(hashes_hbm, wts_hbm, table_hbm, out_hbm,
          idx0, wts0, idx1, wts1, idx2, wts2, rows0, rows1, outb_v,
          semi0, semi1, semi2, sem0, sem1):
    wid = lax.axis_index("s") * NC + lax.axis_index("c")
    idx = (idx0, idx1, idx2)
    wts = (wts0, wts1, wts2)
    semi = (semi0, semi1, semi2)
    rows = (rows0, rows1)
    sem = (sem0, sem1)

    def load_start(g, q):
        base_g = (wid * RW + g * CB) * HIST
        pltpu.async_copy(hashes_hbm.at[pl.ds(base_g, GC)], idx[q], semi[q])
        pltpu.async_copy(wts_hbm.at[pl.ds(base_g, GC)],
                         wts[q].at[pl.ds(0, GC)], semi[q])

    def gather_start(g, q, p):
        base_g = (wid * RW + g * CB) * HIST
        pltpu.make_async_copy(hashes_hbm.at[pl.ds(base_g, GC)], idx[q],
                              semi[q]).wait()
        pltpu.make_async_copy(wts_hbm.at[pl.ds(base_g, GC)],
                              wts[q].at[pl.ds(0, GC)], semi[q]).wait()

        def wm_body(j, c):
            iv = idx[q][pl.ds(j * L, L)]
            wv = wts[q][pl.ds(j * L, L)]
            wts[q][pl.ds(j * L, L)] = jnp.where(iv == PADDING_IDX, 0.0, wv)
            return c

        lax.fori_loop(0, GC // L, wm_body, 0)
        pltpu.async_copy(table_hbm.at[idx[q]], rows[p], sem[p])

    def consume(g, q, p):
        pltpu.make_async_copy(table_hbm.at[idx[q]], rows[p], sem[p]).wait()
        rv, wv = rows[p], wts[q]

        def row_body(b, c):
            r0 = b * HIST

            def k_body(k, acc):
                a0, a1, b0, b1 = acc
                rk = r0 + k * L
                w16 = wv[pl.ds(rk, L)]
                for j in range(0, L, 2):
                    w = w16[j]
                    a0 = a0 + w * rv[rk + j, pl.ds(0, L)]
                    a1 = a1 + w * rv[rk + j, pl.ds(L, L)]
                    w2 = w16[j + 1]
                    b0 = b0 + w2 * rv[rk + j + 1, pl.ds(0, L)]
                    b1 = b1 + w2 * rv[rk + j + 1, pl.ds(L, L)]
                return (a0, a1, b0, b1)

            z = jnp.zeros((L,), jnp.float32)
            a0, a1, b0, b1 = lax.fori_loop(0, HIST // L, k_body, (z, z, z, z))
            rt = r0 + (HIST // L) * L
            wt16 = wv[pl.ds(rt, L)]
            a0 = a0 + wt16[0] * rv[rt, pl.ds(0, L)]
            a1 = a1 + wt16[0] * rv[rt, pl.ds(L, L)]
            b0 = b0 + wt16[1] * rv[rt + 1, pl.ds(0, L)]
            b1 = b1 + wt16[1] * rv[rt + 1, pl.ds(L, L)]
            outb_v[b, pl.ds(0, L)] = a0 + b0
            outb_v[b, pl.ds(L, L)] = a1 + b1
            return c

        lax.fori_loop(0, CB, row_body, 0)
        base_b = wid * RW + g * CB
        pltpu.sync_copy(outb_v, out_hbm.at[pl.ds(base_b, CB)])

    load_start(0, 0)
    load_start(1, 1)
    gather_start(0, 0, 0)

    def outer(gb, c):
        for u in range(6):
            g = 6 * gb + u

            @pl.when(g + 2 < NCHUNK)
            def _():
                load_start(g + 2, (u + 2) % 3)

            @pl.when(g + 1 < NCHUNK)
            def _():
                gather_start(g + 1, (u + 1) % 3, (u + 1) % 2)

            @pl.when(g < NCHUNK)
            def _():
                consume(g, u % 3, u % 2)
        return c

    lax.fori_loop(0, (NCHUNK + 5) // 6, outer, 0)


@jax.jit
def kernel(hashes, weights, table):
    hashes_flat = hashes.astype(jnp.int32).reshape(B * HIST)
    weights_flat = weights.reshape(B * HIST)
    mesh = plsc.VectorSubcoreMesh(core_axis_name="c", subcore_axis_name="s")

    t_flat = pl.kernel(
        _tbody,
        out_type=jax.ShapeDtypeStruct((NUM_EMBEDDINGS * D,), jnp.float32),
        mesh=mesh,
        scratch_types=[
            pltpu.VMEM((D, SB), jnp.float32),
            pltpu.VMEM((D, SB), jnp.float32),
            pltpu.VMEM((SB * D,), jnp.float32),
            pltpu.VMEM((SB * D,), jnp.float32),
            pltpu.SemaphoreType.DMA,
            pltpu.SemaphoreType.DMA,
            pltpu.SemaphoreType.DMA,
            pltpu.SemaphoreType.DMA,
            pltpu.VMEM((D, TAIL_H), jnp.float32),
            pltpu.VMEM((TAIL_H * D,), jnp.float32),
        ],
        compiler_params=pltpu.CompilerParams(use_tc_tiling_on_sc=True,
                                             needs_layout_passes=False),
    )(table.T)
    t_rm = t_flat.reshape(NUM_EMBEDDINGS, D)

    run = pl.kernel(
        _body,
        out_type=jax.ShapeDtypeStruct((B, D), jnp.float32),
        mesh=mesh,
        scratch_types=[
            pltpu.VMEM((GC,), jnp.int32),
            pltpu.VMEM((GC + L,), jnp.float32),
            pltpu.VMEM((GC,), jnp.int32),
            pltpu.VMEM((GC + L,), jnp.float32),
            pltpu.VMEM((GC,), jnp.int32),
            pltpu.VMEM((GC + L,), jnp.float32),
            pltpu.VMEM((GC, D), jnp.float32),
            pltpu.VMEM((GC, D), jnp.float32),
            pltpu.VMEM((CB, D), jnp.float32),
            pltpu.SemaphoreType.DMA,
            pltpu.SemaphoreType.DMA,
            pltpu.SemaphoreType.DMA,
            pltpu.SemaphoreType.DMA,
            pltpu.SemaphoreType.DMA,
        ],
        compiler_params=pltpu.CompilerParams(use_tc_tiling_on_sc=False),
    )
    return run(hashes_flat, weights_flat, t_rm)
